# baseline (speedup 1.0000x reference)
import functools

import jax
import jax.numpy as jnp
from jax import lax
from jax.experimental import pallas as pl
from jax.experimental.pallas import tpu as pltpu
from jax.experimental.pallas import tpu_sc as plsc

N = 10000
E = 320000
IN_DIM = 128
D = 64
DH = D // 2
G = 64

NC = 2
NS = 16
NW = NC * NS
CHUNK = 128
HCH = 80
SCH = 160
EPP = SCH * CHUNK
E_PAD = EPP * NS
N_PAD = 10112
RPT = N_PAD // NS
NBUF = 10
HGRP = HCH // NBUF
SGRP = SCH // NBUF

_mesh = plsc.VectorSubcoreMesh(core_axis_name="c", subcore_axis_name="s")
_sc_params = pltpu.CompilerParams(use_tc_tiling_on_sc=False)


@functools.partial(
    pl.kernel,
    out_type=jax.ShapeDtypeStruct((NC * N_PAD, 16), jnp.float32),
    mesh=_mesh,
    scratch_types=[
        pltpu.VMEM((HCH, CHUNK), jnp.int32),
        pltpu.VMEM((CHUNK, 16), jnp.float32),
        pltpu.VMEM_SHARED((N_PAD, 16), jnp.float32),
    ] + [pltpu.SemaphoreType.DMA] * NBUF,
    compiler_params=_sc_params,
)
def _sc_hist(dst2_hbm, ones_hbm, zeros_hbm, out_hbm, dst2_v, ones_v, acc_sh,
             *ssems):
    c = lax.axis_index("c")
    s = lax.axis_index("s")
    tid = s * NC + c
    pltpu.sync_copy(dst2_hbm.at[pl.ds(tid * HCH, HCH)], dst2_v)
    pltpu.sync_copy(zeros_hbm.at[pl.ds(s * RPT, RPT)],
                    acc_sh.at[pl.ds(s * RPT, RPT)])
    pltpu.sync_copy(ones_hbm, ones_v)
    plsc.subcore_barrier()

    def group(g, carry):
        for b in range(NBUF):
            i = g * NBUF + b
            pltpu.async_copy(ones_v, acc_sh.at[dst2_v.at[i]], ssems[b],
                             add=True)
        for b in range(NBUF):
            pltpu.make_async_copy(ones_v, acc_sh.at[dst2_v.at[b]],
                                  ssems[b]).wait()
        return carry

    lax.fori_loop(0, HGRP, group, 0)
    plsc.subcore_barrier()
    pltpu.sync_copy(acc_sh.at[pl.ds(s * RPT, RPT)],
                    out_hbm.at[pl.ds(c * N_PAD + s * RPT, RPT)])


@functools.partial(
    pl.kernel,
    out_type=jax.ShapeDtypeStruct((NC * N_PAD, DH), jnp.float32),
    mesh=_mesh,
    scratch_types=[
        pltpu.VMEM((SCH, CHUNK), jnp.int32),
        pltpu.VMEM((SCH, CHUNK), jnp.int32),
        pltpu.VMEM((NBUF, CHUNK, DH), jnp.float32),
        pltpu.VMEM_SHARED((N_PAD, DH), jnp.float32),
        pltpu.VMEM_SHARED((N_PAD, DH), jnp.float32),
    ] + [pltpu.SemaphoreType.DMA] * (2 * NBUF),
    compiler_params=_sc_params,
)
def _sc_segsum(g2_hbm, src2_hbm, dst2_hbm, out_hbm,
               src2_v, dst2_v, rows_v, acc_sh, g_sh, *sems):
    gsems = sems[:NBUF]
    ssems = sems[NBUF:]
    c = lax.axis_index("c")
    s = lax.axis_index("s")
    pltpu.sync_copy(src2_hbm.at[pl.ds(s * SCH, SCH)], src2_v)
    pltpu.sync_copy(dst2_hbm.at[pl.ds(s * SCH, SCH)], dst2_v)
    pltpu.sync_copy(g2_hbm.at[pl.ds(c * N_PAD + s * RPT, RPT)],
                    acc_sh.at[pl.ds(s * RPT, RPT)])
    pltpu.sync_copy(g2_hbm.at[pl.ds(c * N_PAD + s * RPT, RPT)],
                    g_sh.at[pl.ds(s * RPT, RPT)])
    plsc.subcore_barrier()

    for b in range(NBUF):
        pltpu.async_copy(g_sh.at[src2_v.at[b]], rows_v.at[b], gsems[b])

    def group(g, carry):
        for b in range(NBUF):
            i = g * NBUF + b
            pltpu.make_async_copy(g_sh.at[src2_v.at[b]], rows_v.at[b],
                                  gsems[b]).wait()
            pltpu.async_copy(rows_v.at[b], acc_sh.at[dst2_v.at[i]], ssems[b],
                             add=True)
        for b in range(NBUF):
            pltpu.make_async_copy(rows_v.at[b], acc_sh.at[dst2_v.at[b]],
                                  ssems[b]).wait()

            @pl.when(g + 1 < SGRP)
            def _():
                j = (g + 1) * NBUF + b
                pltpu.async_copy(g_sh.at[src2_v.at[j]], rows_v.at[b],
                                 gsems[b])
        return carry

    lax.fori_loop(0, SGRP, group, 0)
    plsc.subcore_barrier()
    pltpu.sync_copy(acc_sh.at[pl.ds(s * RPT, RPT)],
                    out_hbm.at[pl.ds(c * N_PAD + s * RPT, RPT)])


def _tc_matmul_body(x_ref, we_ref, w0_ref, t_ref):
    h = jnp.dot(x_ref[...], we_ref[...], preferred_element_type=jnp.float32)
    for k in range(NC):
        t_ref[k * N_PAD:k * N_PAD + N] = jnp.dot(
            h, w0_ref[:, k * DH:(k + 1) * DH],
            preferred_element_type=jnp.float32)
        t_ref[k * N_PAD + N:(k + 1) * N_PAD] = jnp.zeros(
            (N_PAD - N, DH), jnp.float32)


def _tc_scale_body(t_ref, degp_ref, g0_ref, dinv_ref):
    deg = degp_ref[0:N, 0:1] + degp_ref[N_PAD:N_PAD + N, 0:1] + 1.0
    dinv = lax.rsqrt(deg)
    dinv_ref[...] = dinv
    for k in range(NC):
        g0_ref[k * N_PAD:k * N_PAD + N] = dinv * t_ref[k * N_PAD:k * N_PAD + N]
        g0_ref[k * N_PAD + N:(k + 1) * N_PAD] = jnp.zeros(
            (N_PAD - N, DH), jnp.float32)


def _bn_half(s_ref, dinv, b, gam, bet, k):
    u = dinv * s_ref[k * N_PAD:k * N_PAD + N] + b[:, k * DH:(k + 1) * DH]
    mu = jnp.mean(u, axis=0, keepdims=True)
    var = jnp.mean((u - mu) ** 2, axis=0, keepdims=True)
    return ((u - mu) * lax.rsqrt(var + 1e-5) * gam[:, k * DH:(k + 1) * DH]
            + bet[:, k * DH:(k + 1) * DH])


def _tc_mid_body(s_ref, dinv_ref, b_ref, gam_ref, bet_ref,
                 w1_ref, g1_ref):
    dinv = dinv_ref[...]
    b = b_ref[...][None, :]
    gam = gam_ref[...][None, :]
    bet = bet_ref[...][None, :]
    hs = [jnp.maximum(_bn_half(s_ref, dinv, b, gam, bet, k), 0.0)
          for k in range(NC)]
    for k in range(NC):
        t = sum(jnp.dot(hs[j], w1_ref[j * DH:(j + 1) * DH, k * DH:(k + 1) * DH],
                        preferred_element_type=jnp.float32)
                for j in range(NC))
        g1_ref[k * N_PAD:k * N_PAD + N] = dinv * t
        g1_ref[k * N_PAD + N:(k + 1) * N_PAD] = jnp.zeros((N_PAD - N, DH), jnp.float32)


def _tc_final_body(s_ref, dinv_ref, b_ref, gam_ref, bet_ref,
                   batch_ref, h_ref, pool_ref):
    dinv = dinv_ref[...]
    b = b_ref[...][None, :]
    gam = gam_ref[...][None, :]
    bet = bet_ref[...][None, :]
    gids = lax.broadcasted_iota(jnp.int32, (G, N), 0)
    onehot = (gids == batch_ref[...]).astype(jnp.float32)
    for k in range(NC):
        h = _bn_half(s_ref, dinv, b, gam, bet, k)
        h_ref[:, k * DH:(k + 1) * DH] = h
        pool_ref[:, k * DH:(k + 1) * DH] = jnp.dot(
            onehot, h, preferred_element_type=jnp.float32)


def kernel(x, edge_index, batch, W_embed, W0, b0, gamma0, beta0,
           W1, b1, gamma1, beta1):
    src = edge_index[0]
    dst = edge_index[1]
    ppt = EPP - E // NS
    dummy_src = jnp.zeros((NS, ppt), jnp.int32)
    dummy_dst = jnp.broadcast_to(
        N + (jnp.arange(ppt, dtype=jnp.int32) % (N_PAD - N)), (NS, ppt))
    src2 = jnp.concatenate([src.reshape(NS, E // NS), dummy_src],
                           axis=1).reshape(NS * SCH, CHUNK)
    dst2 = jnp.concatenate([dst.reshape(NS, E // NS), dummy_dst],
                           axis=1).reshape(NS * SCH, CHUNK)
    ones16 = jnp.ones((CHUNK, 16), jnp.float32)
    zeros16 = jnp.zeros((N_PAD, 16), jnp.float32)

    degp = _sc_hist(dst2, ones16, zeros16)

    traw = pl.pallas_call(
        _tc_matmul_body,
        out_shape=jax.ShapeDtypeStruct((NC * N_PAD, DH), jnp.float32),
    )(x, W_embed, W0)
    g0, dinv = pl.pallas_call(
        _tc_scale_body,
        out_shape=(jax.ShapeDtypeStruct((NC * N_PAD, DH), jnp.float32),
                   jax.ShapeDtypeStruct((N, 1), jnp.float32)),
    )(traw, degp)

    s0 = _sc_segsum(g0, src2, dst2)

    g1 = pl.pallas_call(
        _tc_mid_body,
        out_shape=jax.ShapeDtypeStruct((NC * N_PAD, DH), jnp.float32),
    )(s0, dinv, b0, gamma0, beta0, W1)

    s1 = _sc_segsum(g1, src2, dst2)

    h, pool = pl.pallas_call(
        _tc_final_body,
        out_shape=(jax.ShapeDtypeStruct((N, D), jnp.float32),
                   jax.ShapeDtypeStruct((G, D), jnp.float32)),
    )(s1, dinv, b1, gamma1, beta1, batch.reshape(1, N))

    return (h, pool)


# speedup vs baseline: 1.0094x; 1.0094x over previous
import functools

import jax
import jax.numpy as jnp
from jax import lax
from jax.experimental import pallas as pl
from jax.experimental.pallas import tpu as pltpu
from jax.experimental.pallas import tpu_sc as plsc

N = 10000
E = 320000
IN_DIM = 128
D = 64
DH = D // 2
G = 64

NC = 2
NS = 16
NW = NC * NS
CHUNK = 128
HCH = 80
SCH = 160
EPP = SCH * CHUNK
E_PAD = EPP * NS
N_PAD = 10112
RPT = N_PAD // NS
NBUF = 10
HGRP = HCH // NBUF
SGRP = SCH // NBUF

_mesh = plsc.VectorSubcoreMesh(core_axis_name="c", subcore_axis_name="s")
_sc_params = pltpu.CompilerParams(use_tc_tiling_on_sc=False)


@functools.partial(
    pl.kernel,
    out_type=jax.ShapeDtypeStruct((NC * N_PAD, 16), jnp.float32),
    mesh=_mesh,
    scratch_types=[
        pltpu.VMEM((HCH, CHUNK), jnp.int32),
        pltpu.VMEM((CHUNK, 16), jnp.float32),
        pltpu.VMEM_SHARED((N_PAD, 16), jnp.float32),
    ] + [pltpu.SemaphoreType.DMA] * NBUF,
    compiler_params=_sc_params,
)
def _sc_hist(dst2_hbm, ones_hbm, zeros_hbm, out_hbm, dst2_v, ones_v, acc_sh,
             *ssems):
    c = lax.axis_index("c")
    s = lax.axis_index("s")
    tid = s * NC + c
    pltpu.sync_copy(dst2_hbm.at[pl.ds(tid * HCH, HCH)], dst2_v)
    pltpu.sync_copy(zeros_hbm.at[pl.ds(s * RPT, RPT)],
                    acc_sh.at[pl.ds(s * RPT, RPT)])
    pltpu.sync_copy(ones_hbm, ones_v)
    plsc.subcore_barrier()

    def group(g, carry):
        for b in range(NBUF):
            i = g * NBUF + b
            pltpu.async_copy(ones_v, acc_sh.at[dst2_v.at[i]], ssems[b],
                             add=True)
        for b in range(NBUF):
            pltpu.make_async_copy(ones_v, acc_sh.at[dst2_v.at[b]],
                                  ssems[b]).wait()
        return carry

    lax.fori_loop(0, HGRP, group, 0)
    plsc.subcore_barrier()
    pltpu.sync_copy(acc_sh.at[pl.ds(s * RPT, RPT)],
                    out_hbm.at[pl.ds(c * N_PAD + s * RPT, RPT)])


@functools.partial(
    pl.kernel,
    out_type=jax.ShapeDtypeStruct((NC * N_PAD, DH), jnp.float32),
    mesh=_mesh,
    scratch_types=[
        pltpu.VMEM((SCH, CHUNK), jnp.int32),
        pltpu.VMEM((SCH, CHUNK), jnp.int32),
        pltpu.VMEM((NBUF, CHUNK, DH), jnp.float32),
        pltpu.VMEM_SHARED((N_PAD, DH), jnp.float32),
        pltpu.VMEM_SHARED((N_PAD, DH), jnp.float32),
    ] + [pltpu.SemaphoreType.DMA] * (2 * NBUF),
    compiler_params=_sc_params,
)
def _sc_segsum(g2_hbm, src2_hbm, dst2_hbm, out_hbm,
               src2_v, dst2_v, rows_v, acc_sh, g_sh, *sems):
    gsems = sems[:NBUF]
    ssems = sems[NBUF:]
    c = lax.axis_index("c")
    s = lax.axis_index("s")
    pltpu.sync_copy(src2_hbm.at[pl.ds(s * SCH, SCH)], src2_v)
    pltpu.sync_copy(dst2_hbm.at[pl.ds(s * SCH, SCH)], dst2_v)
    pltpu.sync_copy(g2_hbm.at[pl.ds(c * N_PAD + s * RPT, RPT)],
                    acc_sh.at[pl.ds(s * RPT, RPT)])
    pltpu.sync_copy(g2_hbm.at[pl.ds(c * N_PAD + s * RPT, RPT)],
                    g_sh.at[pl.ds(s * RPT, RPT)])
    plsc.subcore_barrier()

    for b in range(NBUF):
        pltpu.async_copy(g_sh.at[src2_v.at[b]], rows_v.at[b], gsems[b])

    def group(g, carry):
        for b in range(NBUF):
            i = g * NBUF + b
            pltpu.make_async_copy(g_sh.at[src2_v.at[b]], rows_v.at[b],
                                  gsems[b]).wait()
            pltpu.async_copy(rows_v.at[b], acc_sh.at[dst2_v.at[i]], ssems[b],
                             add=True)
        for b in range(NBUF):
            pltpu.make_async_copy(rows_v.at[b], acc_sh.at[dst2_v.at[b]],
                                  ssems[b]).wait()

            @pl.when(g + 1 < SGRP)
            def _():
                j = (g + 1) * NBUF + b
                pltpu.async_copy(g_sh.at[src2_v.at[j]], rows_v.at[b],
                                 gsems[b])
        return carry

    lax.fori_loop(0, SGRP, group, 0)
    plsc.subcore_barrier()
    pltpu.sync_copy(acc_sh.at[pl.ds(s * RPT, RPT)],
                    out_hbm.at[pl.ds(c * N_PAD + s * RPT, RPT)])


def _tc_embed_body(x_ref, we_ref, w0_ref, degp_ref, g0_ref, dinv_ref):
    deg = degp_ref[0:N, 0:1] + degp_ref[N_PAD:N_PAD + N, 0:1] + 1.0
    dinv = lax.rsqrt(deg)
    dinv_ref[...] = dinv
    h = jnp.dot(x_ref[...], we_ref[...], preferred_element_type=jnp.float32)
    for k in range(NC):
        t = jnp.dot(h, w0_ref[:, k * DH:(k + 1) * DH],
                    preferred_element_type=jnp.float32)
        g0_ref[k * N_PAD:k * N_PAD + N] = dinv * t
        g0_ref[k * N_PAD + N:(k + 1) * N_PAD] = jnp.zeros(
            (N_PAD - N, DH), jnp.float32)


def _bn_half(s_ref, dinv, b, gam, bet, k):
    u = dinv * s_ref[k * N_PAD:k * N_PAD + N] + b[:, k * DH:(k + 1) * DH]
    mu = jnp.mean(u, axis=0, keepdims=True)
    var = jnp.mean((u - mu) ** 2, axis=0, keepdims=True)
    return ((u - mu) * lax.rsqrt(var + 1e-5) * gam[:, k * DH:(k + 1) * DH]
            + bet[:, k * DH:(k + 1) * DH])


def _tc_mid_body(s_ref, dinv_ref, b_ref, gam_ref, bet_ref,
                 w1_ref, g1_ref):
    dinv = dinv_ref[...]
    b = b_ref[...][None, :]
    gam = gam_ref[...][None, :]
    bet = bet_ref[...][None, :]
    hs = [jnp.maximum(_bn_half(s_ref, dinv, b, gam, bet, k), 0.0)
          for k in range(NC)]
    for k in range(NC):
        t = sum(jnp.dot(hs[j], w1_ref[j * DH:(j + 1) * DH, k * DH:(k + 1) * DH],
                        preferred_element_type=jnp.float32)
                for j in range(NC))
        g1_ref[k * N_PAD:k * N_PAD + N] = dinv * t
        g1_ref[k * N_PAD + N:(k + 1) * N_PAD] = jnp.zeros((N_PAD - N, DH), jnp.float32)


def _tc_final_body(s_ref, dinv_ref, b_ref, gam_ref, bet_ref,
                   batch_ref, h_ref, pool_ref):
    dinv = dinv_ref[...]
    b = b_ref[...][None, :]
    gam = gam_ref[...][None, :]
    bet = bet_ref[...][None, :]
    gids = lax.broadcasted_iota(jnp.int32, (G, N), 0)
    onehot = (gids == batch_ref[...]).astype(jnp.float32)
    for k in range(NC):
        h = _bn_half(s_ref, dinv, b, gam, bet, k)
        h_ref[:, k * DH:(k + 1) * DH] = h
        pool_ref[:, k * DH:(k + 1) * DH] = jnp.dot(
            onehot, h, preferred_element_type=jnp.float32)


def kernel(x, edge_index, batch, W_embed, W0, b0, gamma0, beta0,
           W1, b1, gamma1, beta1):
    src = edge_index[0]
    dst = edge_index[1]
    ppt = EPP - E // NS
    dummy_src = jnp.zeros((NS, ppt), jnp.int32)
    dummy_dst = jnp.broadcast_to(
        N + (jnp.arange(ppt, dtype=jnp.int32) % (N_PAD - N)), (NS, ppt))
    src2 = jnp.concatenate([src.reshape(NS, E // NS), dummy_src],
                           axis=1).reshape(NS * SCH, CHUNK)
    dst2 = jnp.concatenate([dst.reshape(NS, E // NS), dummy_dst],
                           axis=1).reshape(NS * SCH, CHUNK)
    ones16 = jnp.ones((CHUNK, 16), jnp.float32)
    zeros16 = jnp.zeros((N_PAD, 16), jnp.float32)

    degp = _sc_hist(dst2, ones16, zeros16)

    g0, dinv = pl.pallas_call(
        _tc_embed_body,
        out_shape=(jax.ShapeDtypeStruct((NC * N_PAD, DH), jnp.float32),
                   jax.ShapeDtypeStruct((N, 1), jnp.float32)),
    )(x, W_embed, W0, degp)

    s0 = _sc_segsum(g0, src2, dst2)

    g1 = pl.pallas_call(
        _tc_mid_body,
        out_shape=jax.ShapeDtypeStruct((NC * N_PAD, DH), jnp.float32),
    )(s0, dinv, b0, gamma0, beta0, W1)

    s1 = _sc_segsum(g1, src2, dst2)

    h, pool = pl.pallas_call(
        _tc_final_body,
        out_shape=(jax.ShapeDtypeStruct((N, D), jnp.float32),
                   jax.ShapeDtypeStruct((G, D), jnp.float32)),
    )(s1, dinv, b1, gamma1, beta1, batch.reshape(1, N))

    return (h, pool)
